# Initial kernel scaffold; baseline (speedup 1.0000x reference)
import jax
import jax.numpy as jnp
from jax import lax
from jax.experimental import pallas as pl
from jax.experimental.pallas import tpu as pltpu
from jax.experimental.pallas import tpu_sc as plsc

F32 = jnp.float32

_NC = 2
_NS = 16
_CHUNK = 128
_DEGW = 16


def _sc_mesh():
    return plsc.VectorSubcoreMesh(core_axis_name="c", subcore_axis_name="s")


def _make_deg_kernel(nchunks, n_nodes):
    rows = n_nodes // _NS
    cpc = nchunks // _NC

    def body(dst_hbm, ones_hbm, out_hbm, ones_v, idx_v, zrow_v, deg_sh):
        c = lax.axis_index("c")
        s = lax.axis_index("s")
        zv = jnp.zeros((16,), F32)

        @pl.loop(0, rows)
        def _(r):
            zrow_v[r, pl.ds(0, 16)] = zv

        pltpu.sync_copy(zrow_v, deg_sh.at[pl.ds(s * rows, rows)])
        pltpu.sync_copy(ones_hbm, ones_v)
        plsc.subcore_barrier()

        @pl.loop(c * cpc + s, (c + 1) * cpc, step=_NS)
        def _(j):
            pltpu.sync_copy(dst_hbm.at[j], idx_v)
            pltpu.sync_copy(ones_v, deg_sh.at[idx_v], add=True)

        plsc.subcore_barrier()
        pltpu.sync_copy(deg_sh.at[pl.ds(s * rows, rows)],
                        out_hbm.at[c, pl.ds(s * rows, rows)])

    return pl.kernel(
        body,
        out_type=jax.ShapeDtypeStruct((_NC, n_nodes, _DEGW), F32),
        mesh=_sc_mesh(),
        scratch_types=[
            pltpu.VMEM((_CHUNK, _DEGW), F32),
            pltpu.VMEM((_CHUNK,), jnp.int32),
            pltpu.VMEM((rows, _DEGW), F32),
            pltpu.VMEM_SHARED((n_nodes, _DEGW), F32),
        ],
    )


def _make_edge_kernel(nchunks, n_nodes, d):
    rows = n_nodes // _NS
    zrows = 125
    cpc = nchunks // _NC

    def body(hs_hbm, src_hbm, dst_hbm, out_hbm,
             idx_s, idx_d, rows_v, zbuf_v, acc_sh):
        c = lax.axis_index("c")
        s = lax.axis_index("s")
        zv = jnp.zeros((16,), F32)

        @pl.loop(0, zrows)
        def _(r):
            @pl.loop(0, d, step=16)
            def _(c0):
                zbuf_v[r, pl.ds(c0, 16)] = zv

        @pl.loop(0, rows // zrows)
        def _(k):
            pltpu.sync_copy(zbuf_v, acc_sh.at[pl.ds(s * rows + k * zrows, zrows)])

        plsc.subcore_barrier()

        @pl.loop(c * cpc + s, (c + 1) * cpc, step=_NS)
        def _(j):
            pltpu.sync_copy(src_hbm.at[j], idx_s)
            pltpu.sync_copy(dst_hbm.at[j], idx_d)
            pltpu.sync_copy(hs_hbm.at[idx_s], rows_v)
            pltpu.sync_copy(rows_v, acc_sh.at[idx_d], add=True)

        plsc.subcore_barrier()
        pltpu.sync_copy(acc_sh.at[pl.ds(s * rows, rows)],
                        out_hbm.at[c, pl.ds(s * rows, rows)])

    return pl.kernel(
        body,
        out_type=jax.ShapeDtypeStruct((_NC, n_nodes, d), F32),
        mesh=_sc_mesh(),
        scratch_types=[
            pltpu.VMEM((_CHUNK,), jnp.int32),
            pltpu.VMEM((_CHUNK,), jnp.int32),
            pltpu.VMEM((_CHUNK, d), F32),
            pltpu.VMEM((zrows, d), F32),
            pltpu.VMEM_SHARED((n_nodes, d), F32),
        ],
    )


def _mm1_body(x_ref, w1_ref, b1_ref, wc_ref, h_ref):
    t = lax.dot_general(x_ref[...], w1_ref[...], (((1,), (1,)), ((), ())),
                        preferred_element_type=F32)
    t = jnp.maximum(t + b1_ref[...], 0.0)
    h_ref[...] = lax.dot_general(t, wc_ref[...], (((1,), (1,)), ((), ())),
                                 preferred_element_type=F32)


def _scale_body(h_ref, dp_ref, hs_ref):
    dp = dp_ref[...]
    deg = (jnp.sum(dp[0], axis=1, keepdims=True)
           + jnp.sum(dp[1], axis=1, keepdims=True) + 1.0)
    hs_ref[...] = h_ref[...] * lax.rsqrt(deg)


def _final_body(ap_ref, hs_ref, dp_ref, bc_ref, wo_ref, bo_ref, y_ref):
    dp = dp_ref[...]
    deg = (jnp.sum(dp[0], axis=1, keepdims=True)
           + jnp.sum(dp[1], axis=1, keepdims=True) + 1.0)
    dinv = lax.rsqrt(deg)
    hs = hs_ref[...]
    acc = ap_ref[0] + ap_ref[1]
    a = jnp.maximum((acc + hs) * dinv + bc_ref[...], 0.0)
    y_ref[...] = lax.dot_general(a, wo_ref[...], (((1,), (1,)), ((), ())),
                                 preferred_element_type=F32) + bo_ref[...]


def _row_spec(br, d):
    return pl.BlockSpec((br, d), lambda i: (i, 0))


def _full_spec(shape):
    nd = len(shape)
    return pl.BlockSpec(shape, lambda i: (0,) * nd)


def kernel(x, edge_index, W1, b1, Wc, bc, Wo, bo):
    n, d = x.shape
    e = edge_index.shape[1]
    nchunks = e // _CHUNK
    br = 2000
    grid = (n // br,)

    src2d = edge_index[0].reshape(nchunks, _CHUNK)
    dst2d = edge_index[1].reshape(nchunks, _CHUNK)
    ones = jnp.ones((_CHUNK, _DEGW), F32)
    b1r = b1.reshape(1, d)
    bcr = bc.reshape(1, d)
    bor = bo.reshape(1, d)

    deg_part = _make_deg_kernel(nchunks, n)(dst2d, ones)

    h = pl.pallas_call(
        _mm1_body,
        grid=grid,
        in_specs=[_row_spec(br, d), _full_spec((d, d)), _full_spec((1, d)),
                  _full_spec((d, d))],
        out_specs=_row_spec(br, d),
        out_shape=jax.ShapeDtypeStruct((n, d), F32),
    )(x, W1, b1r, Wc)

    hs = pl.pallas_call(
        _scale_body,
        grid=grid,
        in_specs=[_row_spec(br, d),
                  pl.BlockSpec((_NC, br, _DEGW), lambda i: (0, i, 0))],
        out_specs=_row_spec(br, d),
        out_shape=jax.ShapeDtypeStruct((n, d), F32),
    )(h, deg_part)

    acc_part = _make_edge_kernel(nchunks, n, d)(hs, src2d, dst2d)

    y = pl.pallas_call(
        _final_body,
        grid=grid,
        in_specs=[pl.BlockSpec((_NC, br, d), lambda i: (0, i, 0)),
                  _row_spec(br, d),
                  pl.BlockSpec((_NC, br, _DEGW), lambda i: (0, i, 0)),
                  _full_spec((1, d)), _full_spec((d, d)), _full_spec((1, d))],
        out_specs=_row_spec(br, d),
        out_shape=jax.ShapeDtypeStruct((n, d), F32),
    )(acc_part, hs, deg_part, bcr, Wo, bor)

    return y


# speedup vs baseline: 22.2115x; 22.2115x over previous
import jax
import jax.numpy as jnp
from jax import lax
from jax.experimental import pallas as pl
from jax.experimental.pallas import tpu as pltpu
from jax.experimental.pallas import tpu_sc as plsc

F32 = jnp.float32

_NC = 2
_NS = 16
_CHUNK = 128
_NPAD = 16384


def _sc_mesh():
    return plsc.VectorSubcoreMesh(core_axis_name="c", subcore_axis_name="s")


def _make_deg_kernel(nchunks, n_nodes):
    epw = _NPAD // _NS
    rows_w = epw // 128
    cpc = nchunks // _NC

    def body(dst_hbm, out_hbm, ones_v, idx_v, t1d, t2d, sh):
        c = lax.axis_index("c")
        s = lax.axis_index("s")
        base = pl.multiple_of(s * epw, 8)

        @pl.loop(0, epw, step=16)
        def _(r):
            t1d[pl.ds(r, 16)] = jnp.zeros((16,), F32)

        @pl.loop(0, _CHUNK, step=16)
        def _(r):
            ones_v[pl.ds(r, 16)] = jnp.ones((16,), F32)

        pltpu.sync_copy(t1d, sh.at[pl.ds(base, epw)])
        plsc.subcore_barrier()

        @pl.loop(c * cpc + s, (c + 1) * cpc, step=_NS)
        def _(j):
            pltpu.sync_copy(dst_hbm.at[j], idx_v)
            pltpu.sync_copy(ones_v, sh.at[idx_v], add=True)

        plsc.subcore_barrier()
        pltpu.sync_copy(sh.at[pl.ds(base, epw)], t1d)

        @pl.loop(0, rows_w)
        def _(r):
            @pl.loop(0, 128, step=16)
            def _(c0):
                t2d[r, pl.ds(c0, 16)] = t1d[pl.ds(r * 128 + c0, 16)]

        row0 = pl.multiple_of((c * _NPAD + s * epw) // 128, 8)
        pltpu.sync_copy(t2d, out_hbm.at[pl.ds(row0, rows_w)])

    return pl.kernel(
        body,
        out_type=jax.ShapeDtypeStruct((_NC * _NPAD // 128, 128), F32),
        mesh=_sc_mesh(),
        scratch_types=[
            pltpu.VMEM((_CHUNK,), F32),
            pltpu.VMEM((_CHUNK,), jnp.int32),
            pltpu.VMEM((epw,), F32),
            pltpu.VMEM((rows_w, 128), F32),
            pltpu.VMEM_SHARED((_NPAD,), F32),
        ],
    )


def _make_edge_kernel(nchunks, n_nodes, d):
    step = (n_nodes // _NS) & ~7
    rw = n_nodes - (_NS - 1) * step
    zr = 80
    cpc = nchunks // _NC

    def body(hs_hbm, src_hbm, dst_hbm, out_hbm,
             idx_s, idx_d, rows_v, zb, sh):
        c = lax.axis_index("c")
        s = lax.axis_index("s")
        base = pl.multiple_of(s * step, 8)
        zv = jnp.zeros((16,), F32)

        @pl.loop(0, zr)
        def _(r):
            @pl.loop(0, d, step=16)
            def _(c0):
                zb[r, pl.ds(c0, 16)] = zv

        @pl.loop(0, rw // zr)
        def _(k):
            pltpu.sync_copy(zb, sh.at[pl.ds(pl.multiple_of(base + k * zr, 8), zr)])

        plsc.subcore_barrier()

        @pl.loop(c * cpc + s, (c + 1) * cpc, step=_NS)
        def _(j):
            pltpu.sync_copy(src_hbm.at[j], idx_s)
            pltpu.sync_copy(dst_hbm.at[j], idx_d)
            pltpu.sync_copy(hs_hbm.at[idx_s], rows_v)
            pltpu.sync_copy(rows_v, sh.at[idx_d], add=True)

        plsc.subcore_barrier()
        pltpu.sync_copy(sh.at[pl.ds(base, rw)],
                        out_hbm.at[pl.ds(pl.multiple_of(c * n_nodes + base, 8), rw)])

    return pl.kernel(
        body,
        out_type=jax.ShapeDtypeStruct((_NC * n_nodes, d), F32),
        mesh=_sc_mesh(),
        scratch_types=[
            pltpu.VMEM((_CHUNK,), jnp.int32),
            pltpu.VMEM((_CHUNK,), jnp.int32),
            pltpu.VMEM((_CHUNK, d), F32),
            pltpu.VMEM((zr, d), F32),
            pltpu.VMEM_SHARED((n_nodes, d), F32),
        ],
    )


def _mm1_body(x_ref, w1_ref, b1_ref, wc_ref, h_ref):
    t = lax.dot_general(x_ref[...], w1_ref[...], (((1,), (1,)), ((), ())),
                        preferred_element_type=F32)
    t = jnp.maximum(t + b1_ref[...], 0.0)
    h_ref[...] = lax.dot_general(t, wc_ref[...], (((1,), (1,)), ((), ())),
                                 preferred_element_type=F32)


def _dinv_body(dp_ref, o_ref):
    dp = dp_ref[...]
    npr = _NPAD // 128
    deg = dp[:npr] + dp[npr:] + 1.0
    o_ref[...] = lax.rsqrt(deg)


def _scale_body(h_ref, dv_ref, hs_ref):
    hs_ref[...] = h_ref[...] * dv_ref[...]


def _final_body(acca_ref, accb_ref, hs_ref, dv_ref, bc_ref, wo_ref, bo_ref,
                y_ref):
    acc = acca_ref[...] + accb_ref[...]
    a = jnp.maximum((acc + hs_ref[...]) * dv_ref[...] + bc_ref[...], 0.0)
    y_ref[...] = lax.dot_general(a, wo_ref[...], (((1,), (1,)), ((), ())),
                                 preferred_element_type=F32) + bo_ref[...]


def _row_spec(br, d):
    return pl.BlockSpec((br, d), lambda i: (i, 0))


def _full_spec(shape):
    nd = len(shape)
    return pl.BlockSpec(shape, lambda i: (0,) * nd)


def kernel(x, edge_index, W1, b1, Wc, bc, Wo, bo):
    n, d = x.shape
    e = edge_index.shape[1]
    nchunks = e // _CHUNK
    br = 2000
    grid = (n // br,)

    src2d = edge_index[0].reshape(nchunks, _CHUNK)
    dst2d = edge_index[1].reshape(nchunks, _CHUNK)
    b1r = b1.reshape(1, d)
    bcr = bc.reshape(1, d)
    bor = bo.reshape(1, d)

    deg_pad = _make_deg_kernel(nchunks, n)(dst2d)

    h = pl.pallas_call(
        _mm1_body,
        grid=grid,
        in_specs=[_row_spec(br, d), _full_spec((d, d)), _full_spec((1, d)),
                  _full_spec((d, d))],
        out_specs=_row_spec(br, d),
        out_shape=jax.ShapeDtypeStruct((n, d), F32),
    )(x, W1, b1r, Wc)

    npr = _NPAD // 128
    dinv_pad = pl.pallas_call(
        _dinv_body,
        grid=(1,),
        in_specs=[_full_spec((2 * npr, 128))],
        out_specs=_full_spec((npr, 128)),
        out_shape=jax.ShapeDtypeStruct((npr, 128), F32),
    )(deg_pad)
    dinv_col = dinv_pad.reshape(_NPAD)[:n].reshape(n, 1)

    hs = pl.pallas_call(
        _scale_body,
        grid=grid,
        in_specs=[_row_spec(br, d), _row_spec(br, 1)],
        out_specs=_row_spec(br, d),
        out_shape=jax.ShapeDtypeStruct((n, d), F32),
    )(h, dinv_col)

    acc2 = _make_edge_kernel(nchunks, n, d)(hs, src2d, dst2d)

    nb = n // br
    y = pl.pallas_call(
        _final_body,
        grid=grid,
        in_specs=[_row_spec(br, d),
                  pl.BlockSpec((br, d), lambda i, nb=nb: (i + nb, 0)),
                  _row_spec(br, d),
                  _row_spec(br, 1),
                  _full_spec((1, d)), _full_spec((d, d)), _full_spec((1, d))],
        out_specs=_row_spec(br, d),
        out_shape=jax.ShapeDtypeStruct((n, d), F32),
    )(acc2, acc2, hs, dinv_col, bcr, Wo, bor)

    return y
